# Initial kernel scaffold; baseline (speedup 1.0000x reference)
import jax
import jax.numpy as jnp
from jax.experimental import pallas as pl


def kernel(eu, ei, et, ew, W1_user, W2_user, b_user, v_user, W1_item, W2_item, b_item, v_item, W1_tag, W2_tag, b_tag, v_tag, U, q, p, u_iw_j, u_iw_w, u_tw_j, u_tw_w, i_uw_j, i_uw_w, i_tw_j, i_tw_w, t_uw_j, t_uw_w, t_iw_j, t_iw_w):
    raise NotImplementedError("write your pallas kernel here")


# speedup vs baseline: 1.3134x; 1.3134x over previous
import functools

import jax
import jax.numpy as jnp
from jax import lax
from jax.experimental import pallas as pl
from jax.experimental.pallas import tpu as pltpu
from jax.experimental.pallas import tpu_sc as plsc

N = 10000
D = 128
DW = 16
DA = 128
K = 16
NWT = 100
NPAD = 10240
BN = 256
CH = 64
F32 = jnp.float32

_info = plsc.get_sparse_core_info()
NC = _info.num_cores
NS = _info.num_subcores
NWK = NC * NS
SPAN = NPAD // NWK


def _k1_body(a_ref, b_ref, o_ref):
    o_ref[0] = jnp.dot(a_ref[0], b_ref[0], preferred_element_type=F32)


def _k1(EA, BB):
    return pl.pallas_call(
        _k1_body,
        grid=(3, NPAD // BN),
        in_specs=[
            pl.BlockSpec((1, BN, D), lambda m, n: (m, n, 0)),
            pl.BlockSpec((1, D, 3 * DA), lambda m, n: (m, 0, 0)),
        ],
        out_specs=pl.BlockSpec((1, BN, 3 * DA), lambda m, n: (m, n, 0)),
        out_shape=jax.ShapeDtypeStruct((3, NPAD, 3 * DA), F32),
    )(EA, BB)


def _k2_body(e_ref, w_ref, b_ref, o_ref):
    o_ref[0] = (jnp.dot(e_ref[...], w_ref[0], preferred_element_type=F32)
                + b_ref[0, 0:1, :])


def _k2(ewp, BW, BIAS):
    return pl.pallas_call(
        _k2_body,
        grid=(3,),
        in_specs=[
            pl.BlockSpec((104, D), lambda m: (0, 0)),
            pl.BlockSpec((1, D, DA), lambda m: (m, 0, 0)),
            pl.BlockSpec((1, 8, DA), lambda m: (m, 0, 0)),
        ],
        out_specs=pl.BlockSpec((1, 104, DA), lambda m: (m, 0, 0)),
        out_shape=jax.ShapeDtypeStruct((3, 104, DA), F32),
    )(ewp, BW, BIAS)


def _k3_body(z_ref, u_ref, q_ref, o_ref):
    o_ref[0] = jnp.maximum(
        jnp.dot(z_ref[0], u_ref[...], preferred_element_type=F32)
        + q_ref[0:1, :], 0.0)


def _k3(Z, U, qb):
    return pl.pallas_call(
        _k3_body,
        grid=(9, NPAD // BN),
        in_specs=[
            pl.BlockSpec((1, BN, D), lambda m, n: (m, n, 0)),
            pl.BlockSpec((D, DA), lambda m, n: (0, 0)),
            pl.BlockSpec((8, DA), lambda m, n: (0, 0)),
        ],
        out_specs=pl.BlockSpec((1, BN, DA), lambda m, n: (m, n, 0)),
        out_shape=jax.ShapeDtypeStruct((9, NPAD, DA), F32),
    )(Z, U, qb)


def _k4_body(z_ref, r_ref, p_ref, o_ref):
    z = z_ref[...]
    r = r_ref[...]
    p_row = p_ref[0:1, :]
    x0 = jnp.sum(r[0] * p_row, axis=-1, keepdims=True)
    x1 = jnp.sum(r[1] * p_row, axis=-1, keepdims=True)
    x2 = jnp.sum(r[2] * p_row, axis=-1, keepdims=True)
    m = jnp.maximum(jnp.maximum(x0, x1), x2)
    e0 = jnp.exp(x0 - m)
    e1 = jnp.exp(x1 - m)
    e2 = jnp.exp(x2 - m)
    s = e0 + e1 + e2
    o_ref[0] = (e0 * z[0] + e1 * z[1] + e2 * z[2]) / s


def _k4(Z, R9, pb):
    return pl.pallas_call(
        _k4_body,
        grid=(3, NPAD // BN),
        in_specs=[
            pl.BlockSpec((3, BN, D), lambda o, n: (o, n, 0)),
            pl.BlockSpec((3, BN, DA), lambda o, n: (o, n, 0)),
            pl.BlockSpec((8, DA), lambda o, n: (0, 0)),
        ],
        out_specs=pl.BlockSpec((1, BN, D), lambda o, n: (o, n, 0)),
        out_shape=jax.ShapeDtypeStruct((3, NPAD, D), F32),
    )(Z, R9, pb)


def _reduce_lanes(m_s, vec, op):
    r = vec
    for sh in (8, 4, 2, 1):
        m_s[pl.ds(0, 16)] = r
        r = op(r, m_s[pl.ds(sh, 16)])
    return r[0]


def _sc_one_call(tc_hbm, t_idx, s_hbm, vj_hbm, vw_hbm, o_hbm,
                 tw_s, v_s, vj_s, vw_s, s_s, o_s, rows_a, a_s, m_s, sem_a,
                 base0):
    lane = lax.broadcasted_iota(jnp.int32, (16,), 0)
    t_base = t_idx * 104 * DA
    vv = [v_s[t_idx, pl.ds(dc * 16, 16)] for dc in range(8)]

    def chunk_body(ch, _):
        base = base0 + ch * CH
        pltpu.sync_copy(vj_hbm.at[pl.ds(base, CH)], vj_s)
        pltpu.sync_copy(vw_hbm.at[pl.ds(base * K, CH * K)],
                        vw_s.at[pl.ds(0, CH * K)])
        pltpu.sync_copy(s_hbm.at[pl.ds(base, CH)], s_s)

        def node_body(c, _):
            pltpu.async_copy(tc_hbm.at[vj_s.at[c]], rows_a, sem_a).wait()
            sv = [s_s[c, pl.ds(dc * 16, 16)] for dc in range(8)]

            def k_body(k, xv):
                w = vw_s[pl.ds(c * K + k, 16)][0]
                tw_base = t_base + w * DA
                acc = None
                for dc in range(8):
                    g1 = rows_a[k, pl.ds(dc * 16, 16)]
                    tw = tw_s[pl.ds(tw_base + dc * 16, 16)]
                    term = jnp.maximum(g1 + tw + sv[dc], 0.0) * vv[dc]
                    acc = term if acc is None else acc + term
                xk = _reduce_lanes(m_s, acc, jnp.add)
                return jnp.where(lane == k, xk, xv)

            xv = lax.fori_loop(0, 16, k_body, jnp.zeros((16,), F32))
            m = _reduce_lanes(m_s, xv, jnp.maximum)
            e = jnp.exp(xv - m)
            a = e / _reduce_lanes(m_s, e, jnp.add)
            a_s[pl.ds(0, 16)] = a

            def w_body(k, oc):
                ak = a_s[pl.ds(k, 16)][0]
                return tuple(oc[dc] + rows_a[k, pl.ds(D + dc * 16, 16)] * ak
                             for dc in range(8))

            oc = lax.fori_loop(0, 16, w_body,
                               tuple(jnp.zeros((16,), F32) for _ in range(8)))
            for dc in range(8):
                o_s[c, pl.ds(dc * 16, 16)] = oc[dc]
            return 0

        lax.fori_loop(0, CH, node_body, 0)
        pltpu.sync_copy(o_s, o_hbm.at[pl.ds(base, CH)])
        return 0

    lax.fori_loop(0, SPAN // CH, chunk_body, 0)


def _sc_stage(Tc0, Tc1, Tc2, TW, V3, S_list, vj_list, vw_list):
    mesh = plsc.VectorSubcoreMesh(core_axis_name="c", subcore_axis_name="s")
    out_type = [jax.ShapeDtypeStruct((NPAD, D), F32) for _ in range(6)]
    scratch = [
        pltpu.VMEM((3 * 104 * DA,), F32),
        pltpu.VMEM((3, DA), F32),
        pltpu.VMEM((CH, K), jnp.int32),
        pltpu.VMEM((CH * K + 16,), jnp.int32),
        pltpu.VMEM((CH, DA), F32),
        pltpu.VMEM((CH, D), F32),
        pltpu.VMEM((K, 2 * D), F32),
        pltpu.VMEM((32,), F32),
        pltpu.VMEM((32,), F32),
        pltpu.SemaphoreType.DMA,
    ]

    @functools.partial(pl.kernel, out_type=out_type, mesh=mesh,
                       scratch_types=scratch)
    def sc_kernel(tc0, tc1, tc2, tw_hbm, v_hbm,
                  s1, s2, s3, s4, s5, s6,
                  vj1, vj2, vj3, vj4, vj5, vj6,
                  vw1, vw2, vw3, vw4, vw5, vw6,
                  o1, o2, o3, o4, o5, o6,
                  tw_s, v_s, vj_s, vw_s, s_s, o_s, rows_a, a_s, m_s, sem_a):
        wid = lax.axis_index("s") * NC + lax.axis_index("c")
        base0 = wid * SPAN
        pltpu.sync_copy(tw_hbm, tw_s)
        pltpu.sync_copy(v_hbm, v_s)
        tcs = (tc0, tc1, tc2)
        tids = (0, 1, 2, 1, 2, 0)
        ss = (s1, s2, s3, s4, s5, s6)
        vjs = (vj1, vj2, vj3, vj4, vj5, vj6)
        vws = (vw1, vw2, vw3, vw4, vw5, vw6)
        os_ = (o1, o2, o3, o4, o5, o6)
        for i in range(6):
            _sc_one_call(tcs[tids[i]], tids[i], ss[i], vjs[i], vws[i], os_[i],
                         tw_s, v_s, vj_s, vw_s, s_s, o_s, rows_a, a_s, m_s,
                         sem_a, base0)

    return sc_kernel(Tc0, Tc1, Tc2, TW, V3, *S_list, *vj_list, *vw_list)


def kernel(eu, ei, et, ew, W1_user, W2_user, b_user, v_user, W1_item, W2_item,
           b_item, v_item, W1_tag, W2_tag, b_tag, v_tag, U, q, p,
           u_iw_j, u_iw_w, u_tw_j, u_tw_w, i_uw_j, i_uw_w, i_tw_j, i_tw_w,
           t_uw_j, t_uw_w, t_iw_j, t_iw_w):
    padr = lambda a: jnp.pad(a, ((0, NPAD - N), (0, 0)))
    EA = jnp.stack([padr(eu), padr(ei), padr(et)])
    w1a = lambda W: W[:D]
    w1b = lambda W: W[D:]
    BB = jnp.stack([
        jnp.concatenate([W2_user, w1a(W1_item), w1a(W1_tag)], axis=1),
        jnp.concatenate([W2_item, w1a(W1_user), w1a(W1_tag)], axis=1),
        jnp.concatenate([W2_tag, w1a(W1_user), w1a(W1_item)], axis=1),
    ])
    P = _k1(EA, BB)

    ewp = jnp.concatenate([jnp.zeros((1, DW), F32), ew], axis=0)
    ewp = jnp.pad(ewp, ((0, 3), (0, D - DW)))
    padw = lambda W: jnp.pad(w1b(W), ((0, D - DW), (0, 0)))
    BW = jnp.stack([padw(W1_item), padw(W1_tag), padw(W1_user)])
    BIAS = jnp.stack([jnp.broadcast_to(b_item, (8, DA)),
                      jnp.broadcast_to(b_tag, (8, DA)),
                      jnp.broadcast_to(b_user, (8, DA))])
    TW = _k2(ewp, BW, BIAS)

    zrow = jnp.zeros((1, 2 * D), F32)
    Tc0 = jnp.concatenate(
        [zrow, jnp.concatenate([P[1, :N, 0:D], ei], axis=1)], axis=0)
    Tc1 = jnp.concatenate(
        [zrow, jnp.concatenate([P[2, :N, 0:D], et], axis=1)], axis=0)
    Tc2 = jnp.concatenate(
        [zrow, jnp.concatenate([P[0, :N, 0:D], eu], axis=1)], axis=0)
    S_list = (P[0, :, D:2 * D], P[0, :, 2 * D:], P[1, :, D:2 * D],
              P[1, :, 2 * D:], P[2, :, D:2 * D], P[2, :, 2 * D:])
    V3 = jnp.concatenate([v_item, v_tag, v_user], axis=0)
    vj_list = tuple(padr(a) for a in
                    (u_iw_j, u_tw_j, i_uw_j, i_tw_j, t_uw_j, t_iw_j))
    vw_list = tuple(padr(a).reshape(-1) for a in
                    (u_iw_w, u_tw_w, i_uw_w, i_tw_w, t_uw_w, t_iw_w))

    O = _sc_stage(Tc0, Tc1, Tc2, TW.reshape(-1), V3, S_list, vj_list,
                  vw_list)

    Z = jnp.stack([EA[0], O[0], O[1], O[2], EA[1], O[3], O[4], O[5], EA[2]])
    qb = jnp.broadcast_to(q, (8, DA))
    pb = jnp.broadcast_to(p, (8, DA))
    R9 = _k3(Z, U, qb)
    OUT = _k4(Z, R9, pb)
    return (OUT[0, :N], OUT[1, :N], OUT[2, :N])


# speedup vs baseline: 2.1915x; 1.6686x over previous
import functools

import jax
import jax.numpy as jnp
from jax import lax
from jax.experimental import pallas as pl
from jax.experimental.pallas import tpu as pltpu
from jax.experimental.pallas import tpu_sc as plsc

N = 10000
D = 128
DW = 16
DA = 128
K = 16
NWT = 100
NPAD = 10240
BN = 256
CH = 64
F32 = jnp.float32

_info = plsc.get_sparse_core_info()
NC = _info.num_cores
NS = _info.num_subcores
NWK = NC * NS
SPAN = NPAD // NWK


def _k1_body(a_ref, b_ref, o_ref):
    o_ref[0] = jnp.dot(a_ref[0], b_ref[0], preferred_element_type=F32)


def _k1(EA, BB):
    return pl.pallas_call(
        _k1_body,
        grid=(3, NPAD // BN),
        in_specs=[
            pl.BlockSpec((1, BN, D), lambda m, n: (m, n, 0)),
            pl.BlockSpec((1, D, 3 * DA), lambda m, n: (m, 0, 0)),
        ],
        out_specs=pl.BlockSpec((1, BN, 3 * DA), lambda m, n: (m, n, 0)),
        out_shape=jax.ShapeDtypeStruct((3, NPAD, 3 * DA), F32),
    )(EA, BB)


def _k2_body(e_ref, w_ref, b_ref, o_ref):
    o_ref[0] = (jnp.dot(e_ref[...], w_ref[0], preferred_element_type=F32)
                + b_ref[0, 0:1, :])


def _k2(ewp, BW, BIAS):
    return pl.pallas_call(
        _k2_body,
        grid=(3,),
        in_specs=[
            pl.BlockSpec((104, D), lambda m: (0, 0)),
            pl.BlockSpec((1, D, DA), lambda m: (m, 0, 0)),
            pl.BlockSpec((1, 8, DA), lambda m: (m, 0, 0)),
        ],
        out_specs=pl.BlockSpec((1, 104, DA), lambda m: (m, 0, 0)),
        out_shape=jax.ShapeDtypeStruct((3, 104, DA), F32),
    )(ewp, BW, BIAS)


def _k3_body(z_ref, u_ref, q_ref, o_ref):
    o_ref[0] = jnp.maximum(
        jnp.dot(z_ref[0], u_ref[...], preferred_element_type=F32)
        + q_ref[0:1, :], 0.0)


def _k3(Z, U, qb):
    return pl.pallas_call(
        _k3_body,
        grid=(9, NPAD // BN),
        in_specs=[
            pl.BlockSpec((1, BN, D), lambda m, n: (m, n, 0)),
            pl.BlockSpec((D, DA), lambda m, n: (0, 0)),
            pl.BlockSpec((8, DA), lambda m, n: (0, 0)),
        ],
        out_specs=pl.BlockSpec((1, BN, DA), lambda m, n: (m, n, 0)),
        out_shape=jax.ShapeDtypeStruct((9, NPAD, DA), F32),
    )(Z, U, qb)


def _k4_body(z_ref, r_ref, p_ref, o_ref):
    z = z_ref[...]
    r = r_ref[...]
    p_row = p_ref[0:1, :]
    x0 = jnp.sum(r[0] * p_row, axis=-1, keepdims=True)
    x1 = jnp.sum(r[1] * p_row, axis=-1, keepdims=True)
    x2 = jnp.sum(r[2] * p_row, axis=-1, keepdims=True)
    m = jnp.maximum(jnp.maximum(x0, x1), x2)
    e0 = jnp.exp(x0 - m)
    e1 = jnp.exp(x1 - m)
    e2 = jnp.exp(x2 - m)
    s = e0 + e1 + e2
    o_ref[0] = (e0 * z[0] + e1 * z[1] + e2 * z[2]) / s


def _k4(Z, R9, pb):
    return pl.pallas_call(
        _k4_body,
        grid=(3, NPAD // BN),
        in_specs=[
            pl.BlockSpec((3, BN, D), lambda o, n: (o, n, 0)),
            pl.BlockSpec((3, BN, DA), lambda o, n: (o, n, 0)),
            pl.BlockSpec((8, DA), lambda o, n: (0, 0)),
        ],
        out_specs=pl.BlockSpec((1, BN, D), lambda o, n: (o, n, 0)),
        out_shape=jax.ShapeDtypeStruct((3, NPAD, D), F32),
    )(Z, R9, pb)


def _reduce_lanes(m_s, vec, op):
    r = vec
    for sh in (8, 4, 2, 1):
        m_s[pl.ds(0, 16)] = r
        r = op(r, m_s[pl.ds(sh, 16)])
    return r[0]


def _sc_one_call(tc_hbm, t_idx, s_hbm, vj_hbm, vw_hbm, o_hbm,
                 tw_s, v_s, vj_s, vw_s, s_s, o_s, rows_a, rows_b, a_s, m_s,
                 sem_a, sem_b, base0):
    lane = lax.broadcasted_iota(jnp.int32, (16,), 0)
    t_base = t_idx * 104 * DA
    vv = [v_s[t_idx, pl.ds(dc * 16, 16)] for dc in range(8)]

    def fire(c, buf, sem):
        pltpu.async_copy(tc_hbm.at[vj_s.at[c]], buf, sem)

    def wait(c, buf, sem):
        pltpu.make_async_copy(tc_hbm.at[vj_s.at[c]], buf, sem).wait()

    def compute(c, buf):
        sv = [s_s[c, pl.ds(dc * 16, 16)] for dc in range(8)]

        def k_body(k, xv):
            w = vw_s[pl.ds(c * K + k, 16)][0]
            tw_base = t_base + w * DA
            acc = None
            for dc in range(8):
                g1 = buf[k, pl.ds(dc * 16, 16)]
                tw = tw_s[pl.ds(tw_base + dc * 16, 16)]
                term = jnp.maximum(g1 + tw + sv[dc], 0.0) * vv[dc]
                acc = term if acc is None else acc + term
            xk = _reduce_lanes(m_s, acc, jnp.add)
            return jnp.where(lane == k, xk, xv)

        xv = lax.fori_loop(0, 16, k_body, jnp.zeros((16,), F32))
        m = _reduce_lanes(m_s, xv, jnp.maximum)
        e = jnp.exp(xv - m)
        a = e / _reduce_lanes(m_s, e, jnp.add)
        a_s[pl.ds(0, 16)] = a

        def w_body(k, oc):
            ak = a_s[pl.ds(k, 16)][0]
            return tuple(oc[dc] + buf[k, pl.ds(D + dc * 16, 16)] * ak
                         for dc in range(8))

        oc = lax.fori_loop(0, 16, w_body,
                           tuple(jnp.zeros((16,), F32) for _ in range(8)))
        for dc in range(8):
            o_s[c, pl.ds(dc * 16, 16)] = oc[dc]

    def chunk_body(ch, _):
        base = base0 + ch * CH
        pltpu.sync_copy(vj_hbm.at[pl.ds(base, CH)], vj_s)
        pltpu.sync_copy(vw_hbm.at[pl.ds(base * K, CH * K)],
                        vw_s.at[pl.ds(0, CH * K)])
        pltpu.sync_copy(s_hbm.at[pl.ds(base, CH)], s_s)
        fire(0, rows_a, sem_a)

        def pair_body(i, _):
            c0 = 2 * i
            fire(c0 + 1, rows_b, sem_b)
            wait(c0, rows_a, sem_a)
            compute(c0, rows_a)

            @pl.when(i + 1 < CH // 2)
            def _():
                fire(c0 + 2, rows_a, sem_a)

            wait(c0 + 1, rows_b, sem_b)
            compute(c0 + 1, rows_b)
            return 0

        lax.fori_loop(0, CH // 2, pair_body, 0)
        pltpu.sync_copy(o_s, o_hbm.at[pl.ds(base, CH)])
        return 0

    lax.fori_loop(0, SPAN // CH, chunk_body, 0)


def _sc_stage(Tc0, Tc1, Tc2, TW, V3, S_list, vj_list, vw_list):
    mesh = plsc.VectorSubcoreMesh(core_axis_name="c", subcore_axis_name="s")
    out_type = [jax.ShapeDtypeStruct((NPAD, D), F32) for _ in range(6)]
    scratch = [
        pltpu.VMEM((3 * 104 * DA,), F32),
        pltpu.VMEM((3, DA), F32),
        pltpu.VMEM((CH, K), jnp.int32),
        pltpu.VMEM((CH * K + 16,), jnp.int32),
        pltpu.VMEM((CH, DA), F32),
        pltpu.VMEM((CH, D), F32),
        pltpu.VMEM((K, 2 * D), F32),
        pltpu.VMEM((K, 2 * D), F32),
        pltpu.VMEM((32,), F32),
        pltpu.VMEM((32,), F32),
        pltpu.SemaphoreType.DMA,
        pltpu.SemaphoreType.DMA,
    ]

    @functools.partial(pl.kernel, out_type=out_type, mesh=mesh,
                       scratch_types=scratch)
    def sc_kernel(tc0, tc1, tc2, tw_hbm, v_hbm,
                  s1, s2, s3, s4, s5, s6,
                  vj1, vj2, vj3, vj4, vj5, vj6,
                  vw1, vw2, vw3, vw4, vw5, vw6,
                  o1, o2, o3, o4, o5, o6,
                  tw_s, v_s, vj_s, vw_s, s_s, o_s, rows_a, rows_b, a_s, m_s,
                  sem_a, sem_b):
        wid = lax.axis_index("s") * NC + lax.axis_index("c")
        base0 = wid * SPAN
        pltpu.sync_copy(tw_hbm, tw_s)
        pltpu.sync_copy(v_hbm, v_s)
        tcs = (tc0, tc1, tc2)
        tids = (0, 1, 2, 1, 2, 0)
        ss = (s1, s2, s3, s4, s5, s6)
        vjs = (vj1, vj2, vj3, vj4, vj5, vj6)
        vws = (vw1, vw2, vw3, vw4, vw5, vw6)
        os_ = (o1, o2, o3, o4, o5, o6)
        for i in range(6):
            _sc_one_call(tcs[tids[i]], tids[i], ss[i], vjs[i], vws[i], os_[i],
                         tw_s, v_s, vj_s, vw_s, s_s, o_s, rows_a, rows_b,
                         a_s, m_s, sem_a, sem_b, base0)

    return sc_kernel(Tc0, Tc1, Tc2, TW, V3, *S_list, *vj_list, *vw_list)


def kernel(eu, ei, et, ew, W1_user, W2_user, b_user, v_user, W1_item, W2_item,
           b_item, v_item, W1_tag, W2_tag, b_tag, v_tag, U, q, p,
           u_iw_j, u_iw_w, u_tw_j, u_tw_w, i_uw_j, i_uw_w, i_tw_j, i_tw_w,
           t_uw_j, t_uw_w, t_iw_j, t_iw_w):
    padr = lambda a: jnp.pad(a, ((0, NPAD - N), (0, 0)))
    EA = jnp.stack([padr(eu), padr(ei), padr(et)])
    w1a = lambda W: W[:D]
    w1b = lambda W: W[D:]
    BB = jnp.stack([
        jnp.concatenate([W2_user, w1a(W1_item), w1a(W1_tag)], axis=1),
        jnp.concatenate([W2_item, w1a(W1_user), w1a(W1_tag)], axis=1),
        jnp.concatenate([W2_tag, w1a(W1_user), w1a(W1_item)], axis=1),
    ])
    P = _k1(EA, BB)

    ewp = jnp.concatenate([jnp.zeros((1, DW), F32), ew], axis=0)
    ewp = jnp.pad(ewp, ((0, 3), (0, D - DW)))
    padw = lambda W: jnp.pad(w1b(W), ((0, D - DW), (0, 0)))
    BW = jnp.stack([padw(W1_item), padw(W1_tag), padw(W1_user)])
    BIAS = jnp.stack([jnp.broadcast_to(b_item, (8, DA)),
                      jnp.broadcast_to(b_tag, (8, DA)),
                      jnp.broadcast_to(b_user, (8, DA))])
    TW = _k2(ewp, BW, BIAS)

    zrow = jnp.zeros((1, 2 * D), F32)
    Tc0 = jnp.concatenate(
        [zrow, jnp.concatenate([P[1, :N, 0:D], ei], axis=1)], axis=0)
    Tc1 = jnp.concatenate(
        [zrow, jnp.concatenate([P[2, :N, 0:D], et], axis=1)], axis=0)
    Tc2 = jnp.concatenate(
        [zrow, jnp.concatenate([P[0, :N, 0:D], eu], axis=1)], axis=0)
    S_list = (P[0, :, D:2 * D], P[0, :, 2 * D:], P[1, :, D:2 * D],
              P[1, :, 2 * D:], P[2, :, D:2 * D], P[2, :, 2 * D:])
    V3 = jnp.concatenate([v_item, v_tag, v_user], axis=0)
    vj_list = tuple(padr(a) for a in
                    (u_iw_j, u_tw_j, i_uw_j, i_tw_j, t_uw_j, t_iw_j))
    vw_list = tuple(padr(a).reshape(-1) for a in
                    (u_iw_w, u_tw_w, i_uw_w, i_tw_w, t_uw_w, t_iw_w))

    O = _sc_stage(Tc0, Tc1, Tc2, TW.reshape(-1), V3, S_list, vj_list,
                  vw_list)

    Z = jnp.stack([EA[0], O[0], O[1], O[2], EA[1], O[3], O[4], O[5], EA[2]])
    qb = jnp.broadcast_to(q, (8, DA))
    pb = jnp.broadcast_to(p, (8, DA))
    R9 = _k3(Z, U, qb)
    OUT = _k4(Z, R9, pb)
    return (OUT[0, :N], OUT[1, :N], OUT[2, :N])
